# baseline (device time: 177037 ns/iter reference)
import jax
import jax.numpy as jnp
from jax import lax
from jax.experimental import pallas as pl
from jax.experimental.pallas import tpu as pltpu

N_DEV = 16
SQ = 1024
SKV = 1024
H_LOC = 8
DH = 128
D_LOC = H_LOC * DH
CHUNK = SQ // N_DEV
SCALE = 0.08838834764831843


def _body(x_ref, wq_ref, k_ref, v_ref, wo_ref, out_ref,
          rs_buf, ag_buf, rs_send, rs_recv, ag_send, ag_recv):
    my = lax.axis_index("i")
    right = lax.rem(my + 1, N_DEV)
    left = lax.rem(my + N_DEV - 1, N_DEV)

    barrier_sem = pltpu.get_barrier_semaphore()
    pl.semaphore_signal(barrier_sem, inc=1, device_id=(left,),
                        device_id_type=pl.DeviceIdType.MESH)
    pl.semaphore_signal(barrier_sem, inc=1, device_id=(right,),
                        device_id_type=pl.DeviceIdType.MESH)
    pl.semaphore_wait(barrier_sem, 2)

    f32 = jnp.float32

    q_all = jnp.dot(x_ref[...], wq_ref[...], preferred_element_type=f32)

    rows_b = lax.broadcasted_iota(jnp.int32, (SQ, SKV), 0) // 64
    cols_b = lax.broadcasted_iota(jnp.int32, (SQ, SKV), 1) // 64
    mask = (rows_b == cols_b) | (cols_b == 0) | (lax.rem(rows_b + cols_b, 3) == 0)
    maskadd = jnp.where(mask, 0.0, -1e9).astype(f32)

    ctxs = []
    for h in range(H_LOC):
        qh = q_all[:, h * DH:(h + 1) * DH].astype(jnp.bfloat16)
        s = lax.dot_general(qh, k_ref[h], (((1,), (1,)), ((), ())),
                            preferred_element_type=f32)
        s = s * SCALE + maskadd
        m = jnp.max(s, axis=1, keepdims=True)
        e = jnp.exp(s - m)
        w = (e / jnp.sum(e, axis=1, keepdims=True)).astype(jnp.bfloat16)
        ctxs.append(jnp.dot(w, v_ref[h], preferred_element_type=f32))
    ctx = jnp.concatenate(ctxs, axis=1).astype(jnp.bfloat16)
    out_ref[...] = jnp.dot(ctx, wo_ref[...], preferred_element_type=f32)

    for s_i in range(N_DEV - 1):
        send_c = lax.rem(my - s_i + N_DEV, N_DEV)
        rdma = pltpu.make_async_remote_copy(
            src_ref=out_ref.at[pl.ds(send_c * CHUNK, CHUNK), :],
            dst_ref=rs_buf.at[s_i],
            send_sem=rs_send.at[s_i],
            recv_sem=rs_recv.at[s_i],
            device_id=(right,),
            device_id_type=pl.DeviceIdType.MESH,
        )
        rdma.start()
        rdma.wait()
        recv_c = lax.rem(my - s_i - 1 + N_DEV, N_DEV)
        sl = pl.ds(recv_c * CHUNK, CHUNK)
        out_ref[sl, :] = out_ref[sl, :] + rs_buf[s_i]

    for s_i in range(N_DEV - 1):
        if s_i == 0:
            own_c = lax.rem(my + 1, N_DEV)
            src = out_ref.at[pl.ds(own_c * CHUNK, CHUNK), :]
        else:
            src = ag_buf.at[s_i - 1]
        rdma = pltpu.make_async_remote_copy(
            src_ref=src,
            dst_ref=ag_buf.at[s_i],
            send_sem=ag_send.at[s_i],
            recv_sem=ag_recv.at[s_i],
            device_id=(right,),
            device_id_type=pl.DeviceIdType.MESH,
        )
        rdma.start()
        rdma.wait()
        recv_c = lax.rem(my - s_i + N_DEV, N_DEV)
        out_ref[pl.ds(recv_c * CHUNK, CHUNK), :] = ag_buf[s_i]


def kernel(x, Wq, K_ext, V_ext, Wo):
    i = lax.axis_index("i")
    x2 = x[0].astype(jnp.bfloat16)
    wq_sl = lax.dynamic_slice_in_dim(Wq, i * D_LOC, D_LOC, 1)
    wo_sl = lax.dynamic_slice_in_dim(Wo, i * D_LOC, D_LOC, 0)
    k = jnp.transpose(K_ext[0], (1, 0, 2)).astype(jnp.bfloat16)
    v = jnp.transpose(V_ext[0], (1, 0, 2)).astype(jnp.bfloat16)

    out = pl.pallas_call(
        _body,
        out_shape=jax.ShapeDtypeStruct((SQ, SQ), jnp.float32),
        in_specs=[pl.BlockSpec(memory_space=pltpu.VMEM)] * 5,
        out_specs=pl.BlockSpec(memory_space=pltpu.VMEM),
        scratch_shapes=[
            pltpu.VMEM((N_DEV - 1, CHUNK, SQ), jnp.float32),
            pltpu.VMEM((N_DEV - 1, CHUNK, SQ), jnp.float32),
            pltpu.SemaphoreType.DMA((N_DEV - 1,)),
            pltpu.SemaphoreType.DMA((N_DEV - 1,)),
            pltpu.SemaphoreType.DMA((N_DEV - 1,)),
            pltpu.SemaphoreType.DMA((N_DEV - 1,)),
        ],
        compiler_params=pltpu.CompilerParams(collective_id=0),
    )(x2, wq_sl.astype(jnp.bfloat16), k, v, wo_sl.astype(jnp.bfloat16))
    return out[None]


# device time: 175271 ns/iter; 1.0101x vs baseline; 1.0101x over previous
import jax
import jax.numpy as jnp
from jax import lax
from jax.experimental import pallas as pl
from jax.experimental.pallas import tpu as pltpu

N_DEV = 16
SQ = 1024
SKV = 1024
H_LOC = 8
DH = 128
D_LOC = H_LOC * DH
CHUNK = SQ // N_DEV
SCALE = 0.08838834764831843


HALF = SQ // 2
RCH = HALF // N_DEV


def _body(x_ref, wq_ref, k_ref, v_ref, wo_ref, out_ref,
          rs_buf_r, rs_buf_l, ag_buf_r, ag_buf_l,
          rs_send_r, rs_recv_r, rs_send_l, rs_recv_l,
          ag_send_r, ag_recv_r, ag_send_l, ag_recv_l):
    my = lax.axis_index("i")
    right = lax.rem(my + 1, N_DEV)
    left = lax.rem(my + N_DEV - 1, N_DEV)

    barrier_sem = pltpu.get_barrier_semaphore()
    pl.semaphore_signal(barrier_sem, inc=1, device_id=(left,),
                        device_id_type=pl.DeviceIdType.MESH)
    pl.semaphore_signal(barrier_sem, inc=1, device_id=(right,),
                        device_id_type=pl.DeviceIdType.MESH)
    pl.semaphore_wait(barrier_sem, 2)

    f32 = jnp.float32

    q_all = jnp.dot(x_ref[...], wq_ref[...], preferred_element_type=f32)

    rows_b = lax.broadcasted_iota(jnp.int32, (SQ, SKV), 0) // 64
    cols_b = lax.broadcasted_iota(jnp.int32, (SQ, SKV), 1) // 64
    mask = (rows_b == cols_b) | (cols_b == 0) | (lax.rem(rows_b + cols_b, 3) == 0)
    maskadd = jnp.where(mask, 0.0, -1e9).astype(f32)

    ctxs = []
    for h in range(H_LOC):
        qh = q_all[:, h * DH:(h + 1) * DH].astype(jnp.bfloat16)
        s = lax.dot_general(qh, k_ref[h], (((1,), (1,)), ((), ())),
                            preferred_element_type=f32)
        s = s * SCALE + maskadd
        m = jnp.max(s, axis=1, keepdims=True)
        e = jnp.exp(s - m)
        w = (e / jnp.sum(e, axis=1, keepdims=True)).astype(jnp.bfloat16)
        ctxs.append(jnp.dot(w, v_ref[h], preferred_element_type=f32))
    ctx = jnp.concatenate(ctxs, axis=1).astype(jnp.bfloat16)
    out_ref[...] = jnp.dot(ctx, wo_ref[...], preferred_element_type=f32)

    sends = []
    for s_i in range(N_DEV - 1):
        sc_r = lax.rem(my - s_i + N_DEV, N_DEV)
        rd_r = pltpu.make_async_remote_copy(
            src_ref=out_ref.at[pl.ds(sc_r * RCH, RCH), :],
            dst_ref=rs_buf_r.at[s_i],
            send_sem=rs_send_r.at[s_i], recv_sem=rs_recv_r.at[s_i],
            device_id=(right,), device_id_type=pl.DeviceIdType.MESH)
        sc_l = lax.rem(my + s_i, N_DEV)
        rd_l = pltpu.make_async_remote_copy(
            src_ref=out_ref.at[pl.ds(HALF + sc_l * RCH, RCH), :],
            dst_ref=rs_buf_l.at[s_i],
            send_sem=rs_send_l.at[s_i], recv_sem=rs_recv_l.at[s_i],
            device_id=(left,), device_id_type=pl.DeviceIdType.MESH)
        rd_r.start()
        rd_l.start()
        sends += [rd_r, rd_l]
        rd_r.wait_recv()
        rc_r = lax.rem(my - s_i - 1 + N_DEV, N_DEV)
        sl = pl.ds(rc_r * RCH, RCH)
        out_ref[sl, :] = out_ref[sl, :] + rs_buf_r[s_i]
        rd_l.wait_recv()
        rc_l = lax.rem(my + s_i + 1, N_DEV)
        sl = pl.ds(HALF + rc_l * RCH, RCH)
        out_ref[sl, :] = out_ref[sl, :] + rs_buf_l[s_i]

    for s_i in range(N_DEV - 1):
        if s_i == 0:
            oc_r = lax.rem(my + 1, N_DEV)
            src_r = out_ref.at[pl.ds(oc_r * RCH, RCH), :]
            oc_l = lax.rem(my + N_DEV - 1, N_DEV)
            src_l = out_ref.at[pl.ds(HALF + oc_l * RCH, RCH), :]
        else:
            src_r = ag_buf_r.at[s_i - 1]
            src_l = ag_buf_l.at[s_i - 1]
        rd_r = pltpu.make_async_remote_copy(
            src_ref=src_r, dst_ref=ag_buf_r.at[s_i],
            send_sem=ag_send_r.at[s_i], recv_sem=ag_recv_r.at[s_i],
            device_id=(right,), device_id_type=pl.DeviceIdType.MESH)
        rd_l = pltpu.make_async_remote_copy(
            src_ref=src_l, dst_ref=ag_buf_l.at[s_i],
            send_sem=ag_send_l.at[s_i], recv_sem=ag_recv_l.at[s_i],
            device_id=(left,), device_id_type=pl.DeviceIdType.MESH)
        rd_r.start()
        rd_l.start()
        sends += [rd_r, rd_l]
        rd_r.wait_recv()
        rc_r = lax.rem(my - s_i + N_DEV, N_DEV)
        out_ref[pl.ds(rc_r * RCH, RCH), :] = ag_buf_r[s_i]
        rd_l.wait_recv()
        rc_l = lax.rem(my + s_i, N_DEV)
        out_ref[pl.ds(HALF + rc_l * RCH, RCH), :] = ag_buf_l[s_i]

    for rd in sends:
        rd.wait_send()


def kernel(x, Wq, K_ext, V_ext, Wo):
    i = lax.axis_index("i")
    x2 = x[0].astype(jnp.bfloat16)
    wq_sl = lax.dynamic_slice_in_dim(Wq, i * D_LOC, D_LOC, 1)
    wo_sl = lax.dynamic_slice_in_dim(Wo, i * D_LOC, D_LOC, 0)
    k = jnp.transpose(K_ext[0], (1, 0, 2)).astype(jnp.bfloat16)
    v = jnp.transpose(V_ext[0], (1, 0, 2)).astype(jnp.bfloat16)

    out = pl.pallas_call(
        _body,
        out_shape=jax.ShapeDtypeStruct((SQ, SQ), jnp.float32),
        in_specs=[pl.BlockSpec(memory_space=pltpu.VMEM)] * 5,
        out_specs=pl.BlockSpec(memory_space=pltpu.VMEM),
        scratch_shapes=(
            [pltpu.VMEM((N_DEV - 1, RCH, SQ), jnp.float32)] * 4
            + [pltpu.SemaphoreType.DMA((N_DEV - 1,))] * 8
        ),
        compiler_params=pltpu.CompilerParams(collective_id=0),
    )(x2, wq_sl.astype(jnp.bfloat16), k, v, wo_sl.astype(jnp.bfloat16))
    return out[None]


# device time: 100921 ns/iter; 1.7542x vs baseline; 1.7367x over previous
import os

import jax
import jax.numpy as jnp
from jax import lax
from jax.experimental import pallas as pl
from jax.experimental.pallas import tpu as pltpu

_KPHASE = os.environ.get("KPHASE", "full")

N_DEV = 16
SQ = 1024
SKV = 1024
H_LOC = 8
DH = 128
D_LOC = H_LOC * DH
SCALE = 0.08838834764831843

PCH = 256
STRIP = 128


def _body(x_ref, wq_hbm, k_ref, v_ref, wo_hbm, out_ref,
          wq_vmem, wo_vmem, pbuf_r, pbuf_l, zbuf1, zbuf2,
          wq_sem, wo_sem,
          prs_send_r, prs_recv_r, prs_send_l, prs_recv_l,
          z_send, z_recv,
          pag_send_r, pag_recv_r, pag_send_l, pag_recv_l):
    my = lax.axis_index("i")
    z = my // 4
    p = lax.rem(my, 4)
    b0 = lax.rem(z, 2)
    b1 = z // 2
    p_right = my - p + lax.rem(p + 1, 4)
    p_left = my - p + lax.rem(p + 3, 4)
    z1p = my ^ 4
    z2p = my ^ 8

    wq_dma = pltpu.make_async_copy(
        wq_hbm.at[:, pl.ds(my * D_LOC, D_LOC)], wq_vmem, wq_sem)
    wq_dma.start()
    wo_dma = pltpu.make_async_copy(
        wo_hbm.at[pl.ds(my * D_LOC, D_LOC), :], wo_vmem, wo_sem)
    wo_dma.start()

    barrier_sem = pltpu.get_barrier_semaphore()
    for nbr in (p_left, p_right, z1p, z2p):
        pl.semaphore_signal(barrier_sem, inc=1, device_id=(nbr,),
                            device_id_type=pl.DeviceIdType.MESH)
    pl.semaphore_wait(barrier_sem, 4)

    f32 = jnp.float32

    rows_b = lax.broadcasted_iota(jnp.int32, (SQ, SKV), 0) // 64
    cols_b = lax.broadcasted_iota(jnp.int32, (SQ, SKV), 1) // 64
    mask = (rows_b == cols_b) | (cols_b == 0) | (lax.rem(rows_b + cols_b, 3) == 0)
    maskadd = jnp.where(mask, 0.0, -1e9).astype(f32)

    wq_dma.wait()
    q_all = jnp.dot(x_ref[...], wq_vmem[...].astype(jnp.bfloat16),
                    preferred_element_type=f32)
    ctxs = []
    for h in range(H_LOC):
        hs = slice(h * DH, (h + 1) * DH)
        qh = q_all[:, hs].astype(jnp.bfloat16)
        s = lax.dot_general(qh, k_ref[:, hs], (((1,), (1,)), ((), ())),
                            preferred_element_type=f32)
        e = jnp.exp(s * SCALE + maskadd)
        recip = 1.0 / jnp.sum(e, axis=1, keepdims=True)
        ctx = jnp.dot(e.astype(jnp.bfloat16), v_ref[:, hs],
                      preferred_element_type=f32)
        ctxs.append(ctx * recip)
    ctx = jnp.concatenate(ctxs, axis=1).astype(jnp.bfloat16)
    wo_dma.wait()
    out_ref[...] = jnp.dot(ctx, wo_vmem[...].astype(jnp.bfloat16),
                           preferred_element_type=f32)

    if _KPHASE == "compute":
        return

    sends = []

    def push(src, dst, ssem, rsem, dev):
        r = pltpu.make_async_remote_copy(
            src_ref=src, dst_ref=dst, send_sem=ssem, recv_sem=rsem,
            device_id=(dev,), device_id_type=pl.DeviceIdType.MESH)
        r.start()
        sends.append(r)
        return r

    for s_i in range(3):
        cr = lax.rem(p - s_i + 4, 4)
        rd_r = push(out_ref.at[pl.ds(cr * PCH, STRIP), :], pbuf_r.at[s_i],
                    prs_send_r.at[s_i], prs_recv_r.at[s_i], p_right)
        cl = lax.rem(p + s_i, 4)
        rd_l = push(out_ref.at[pl.ds(cl * PCH + STRIP, STRIP), :], pbuf_l.at[s_i],
                    prs_send_l.at[s_i], prs_recv_l.at[s_i], p_left)
        rd_r.wait_recv()
        rr = lax.rem(p - s_i + 3, 4)
        sl = pl.ds(rr * PCH, STRIP)
        out_ref[sl, :] = out_ref[sl, :] + pbuf_r[s_i]
        rd_l.wait_recv()
        rl = lax.rem(p + s_i + 1, 4)
        sl = pl.ds(rl * PCH + STRIP, STRIP)
        out_ref[sl, :] = out_ref[sl, :] + pbuf_l[s_i]

    base_r = lax.rem(p + 1, 4) * PCH
    base_l = lax.rem(p + 3, 4) * PCH + STRIP
    bases = (base_r, base_l)

    rds = []
    for si, base in enumerate(bases):
        rds.append(push(out_ref.at[pl.ds(base + (1 - b0) * 64, 64), :],
                        zbuf1.at[si], z_send.at[si], z_recv.at[si], z1p))
    for si, base in enumerate(bases):
        rds[si].wait_recv()
        sl = pl.ds(base + b0 * 64, 64)
        out_ref[sl, :] = out_ref[sl, :] + zbuf1[si]
    q_keep = b0 * 64 + b1 * 32
    q_send = b0 * 64 + (1 - b1) * 32
    rds = []
    for si, base in enumerate(bases):
        rds.append(push(out_ref.at[pl.ds(base + q_send, 32), :],
                        zbuf2.at[si], z_send.at[2 + si], z_recv.at[2 + si], z2p))
    for si, base in enumerate(bases):
        rds[si].wait_recv()
        sl = pl.ds(base + q_keep, 32)
        out_ref[sl, :] = out_ref[sl, :] + zbuf2[si]
    rds = []
    for si, base in enumerate(bases):
        sl = pl.ds(base + q_keep, 32)
        rds.append(push(out_ref.at[sl, :], out_ref.at[sl, :],
                        z_send.at[4 + si], z_recv.at[4 + si], z2p))
    for rd in rds:
        rd.wait_recv()
    rds = []
    for si, base in enumerate(bases):
        sl = pl.ds(base + b0 * 64, 64)
        rds.append(push(out_ref.at[sl, :], out_ref.at[sl, :],
                        z_send.at[6 + si], z_recv.at[6 + si], z1p))
    for rd in rds:
        rd.wait_recv()

    for s_i in range(3):
        cr = lax.rem(p + 1 - s_i + 4, 4)
        sl = pl.ds(cr * PCH, STRIP)
        rd_r = push(out_ref.at[sl, :], out_ref.at[sl, :],
                    pag_send_r.at[s_i], pag_recv_r.at[s_i], p_right)
        cl = lax.rem(p + 3 + s_i, 4)
        sl = pl.ds(cl * PCH + STRIP, STRIP)
        rd_l = push(out_ref.at[sl, :], out_ref.at[sl, :],
                    pag_send_l.at[s_i], pag_recv_l.at[s_i], p_left)
        rd_r.wait_recv()
        rd_l.wait_recv()

    for rd in sends:
        rd.wait_send()


def kernel(x, Wq, K_ext, V_ext, Wo):
    x2 = x.reshape(SQ, SQ).astype(jnp.bfloat16)
    k2 = K_ext.reshape(SKV, D_LOC).astype(jnp.bfloat16)
    v2 = V_ext.reshape(SKV, D_LOC).astype(jnp.bfloat16)

    out = pl.pallas_call(
        _body,
        out_shape=jax.ShapeDtypeStruct((SQ, SQ), jnp.float32),
        in_specs=[
            pl.BlockSpec(memory_space=pltpu.VMEM),
            pl.BlockSpec(memory_space=pltpu.MemorySpace.HBM),
            pl.BlockSpec(memory_space=pltpu.VMEM),
            pl.BlockSpec(memory_space=pltpu.VMEM),
            pl.BlockSpec(memory_space=pltpu.MemorySpace.HBM),
        ],
        out_specs=pl.BlockSpec(memory_space=pltpu.VMEM),
        scratch_shapes=(
            [
                pltpu.VMEM((SQ, D_LOC), jnp.float32),
                pltpu.VMEM((D_LOC, SQ), jnp.float32),
                pltpu.VMEM((3, STRIP, SQ), jnp.float32),
                pltpu.VMEM((3, STRIP, SQ), jnp.float32),
                pltpu.VMEM((2, 64, SQ), jnp.float32),
                pltpu.VMEM((2, 32, SQ), jnp.float32),
                pltpu.SemaphoreType.DMA,
                pltpu.SemaphoreType.DMA,
            ]
            + [pltpu.SemaphoreType.DMA((3,))] * 4
            + [pltpu.SemaphoreType.DMA((8,))] * 2
            + [pltpu.SemaphoreType.DMA((3,))] * 4
        ),
        compiler_params=pltpu.CompilerParams(collective_id=0),
    )(x2, Wq, k2, v2, Wo)
    return out.reshape(1, SQ, SQ)


# device time: 74294 ns/iter; 2.3829x vs baseline; 1.3584x over previous
import os

import jax
import jax.numpy as jnp
from jax import lax
from jax.experimental import pallas as pl
from jax.experimental.pallas import tpu as pltpu

_KPHASE = os.environ.get("KPHASE", "full")

N_DEV = 16
SQ = 1024
SKV = 1024
H_LOC = 8
DH = 128
D_LOC = H_LOC * DH
SCALE = 0.08838834764831843

PCH = 256
STRIP = 128


def _body(x_ref, wq_hbm, k_ref, v_ref, wo_hbm, out_ref,
          wq_vmem, wo_vmem, cbuf, pbuf_r, pbuf_l, zbuf1, zbuf2,
          wq_sem, wo_sem,
          prs_send_r, prs_recv_r, prs_send_l, prs_recv_l,
          z_send, z_recv,
          pag_send_r, pag_recv_r, pag_send_l, pag_recv_l):
    my = lax.axis_index("i")
    z = my // 4
    p = lax.rem(my, 4)
    b0 = lax.rem(z, 2)
    b1 = z // 2
    p_right = my - p + lax.rem(p + 1, 4)
    p_left = my - p + lax.rem(p + 3, 4)
    z1p = my ^ 4
    z2p = my ^ 8

    wq_dma = pltpu.make_async_copy(
        wq_hbm.at[:, pl.ds(my * D_LOC, D_LOC)], wq_vmem, wq_sem)
    wq_dma.start()
    wo_dma = pltpu.make_async_copy(
        wo_hbm.at[pl.ds(my * D_LOC, D_LOC), :], wo_vmem, wo_sem)
    wo_dma.start()

    barrier_sem = pltpu.get_barrier_semaphore()
    for nbr in (p_left, p_right, z1p, z2p):
        pl.semaphore_signal(barrier_sem, inc=1, device_id=(nbr,),
                            device_id_type=pl.DeviceIdType.MESH)
    pl.semaphore_wait(barrier_sem, 4)

    f32 = jnp.float32

    rows_b = lax.broadcasted_iota(jnp.int32, (SQ, SKV), 0) // 64
    cols_b = lax.broadcasted_iota(jnp.int32, (SQ, SKV), 1) // 64
    mask = (rows_b == cols_b) | (cols_b == 0) | (lax.rem(rows_b + cols_b, 3) == 0)
    maskadd = jnp.where(mask, 0.0, -1e9).astype(f32)

    wq_dma.wait()
    q_all = jnp.dot(x_ref[...], wq_vmem[...].astype(jnp.bfloat16),
                    preferred_element_type=f32)
    ctxs = []
    for h in range(H_LOC):
        hs = slice(h * DH, (h + 1) * DH)
        qh = q_all[:, hs].astype(jnp.bfloat16)
        s = lax.dot_general(qh, k_ref[:, hs], (((1,), (1,)), ((), ())),
                            preferred_element_type=f32)
        e = jnp.exp(s * SCALE + maskadd)
        recip = 1.0 / jnp.sum(e, axis=1, keepdims=True)
        ctx = jnp.dot(e.astype(jnp.bfloat16), v_ref[:, hs],
                      preferred_element_type=f32)
        ctxs.append(ctx * recip)
    ctx = jnp.concatenate(ctxs, axis=1).astype(jnp.bfloat16)
    wo_dma.wait()
    cbuf[...] = jnp.dot(ctx, wo_vmem[...].astype(jnp.bfloat16),
                        preferred_element_type=f32).astype(jnp.bfloat16)

    if _KPHASE == "compute":
        out_ref[...] = cbuf[...].astype(f32)
        return

    sends = []

    def push(src, dst, ssem, rsem, dev):
        r = pltpu.make_async_remote_copy(
            src_ref=src, dst_ref=dst, send_sem=ssem, recv_sem=rsem,
            device_id=(dev,), device_id_type=pl.DeviceIdType.MESH)
        r.start()
        sends.append(r)
        return r

    for s_i in range(3):
        cr = lax.rem(p - s_i + 4, 4)
        rd_r = push(cbuf.at[pl.ds(cr * PCH, STRIP), :], pbuf_r.at[s_i],
                    prs_send_r.at[s_i], prs_recv_r.at[s_i], p_right)
        cl = lax.rem(p + s_i, 4)
        rd_l = push(cbuf.at[pl.ds(cl * PCH + STRIP, STRIP), :], pbuf_l.at[s_i],
                    prs_send_l.at[s_i], prs_recv_l.at[s_i], p_left)
        rd_r.wait_recv()
        rr = lax.rem(p - s_i + 3, 4)
        sl = pl.ds(rr * PCH, STRIP)
        cbuf[sl, :] = cbuf[sl, :] + pbuf_r[s_i]
        rd_l.wait_recv()
        rl = lax.rem(p + s_i + 1, 4)
        sl = pl.ds(rl * PCH + STRIP, STRIP)
        cbuf[sl, :] = cbuf[sl, :] + pbuf_l[s_i]

    base_r = lax.rem(p + 1, 4) * PCH
    base_l = lax.rem(p + 3, 4) * PCH + STRIP
    bases = (base_r, base_l)

    rds = []
    for si, base in enumerate(bases):
        rds.append(push(cbuf.at[pl.ds(base + (1 - b0) * 64, 64), :],
                        zbuf1.at[si], z_send.at[si], z_recv.at[si], z1p))
    for si, base in enumerate(bases):
        rds[si].wait_recv()
        sl = pl.ds(base + b0 * 64, 64)
        cbuf[sl, :] = cbuf[sl, :] + zbuf1[si]
    q_keep = b0 * 64 + b1 * 32
    q_send = b0 * 64 + (1 - b1) * 32
    rds = []
    for si, base in enumerate(bases):
        rds.append(push(cbuf.at[pl.ds(base + q_send, 32), :],
                        zbuf2.at[si], z_send.at[2 + si], z_recv.at[2 + si], z2p))
    for si, base in enumerate(bases):
        rds[si].wait_recv()
        sl = pl.ds(base + q_keep, 32)
        cbuf[sl, :] = cbuf[sl, :] + zbuf2[si]
    rds = []
    for si, base in enumerate(bases):
        sl = pl.ds(base + q_keep, 32)
        rds.append(push(cbuf.at[sl, :], cbuf.at[sl, :],
                        z_send.at[4 + si], z_recv.at[4 + si], z2p))
    for rd in rds:
        rd.wait_recv()
    rds = []
    for si, base in enumerate(bases):
        sl = pl.ds(base + b0 * 64, 64)
        rds.append(push(cbuf.at[sl, :], cbuf.at[sl, :],
                        z_send.at[6 + si], z_recv.at[6 + si], z1p))
    for rd in rds:
        rd.wait_recv()

    for s_i in range(3):
        cr = lax.rem(p + 1 - s_i + 4, 4)
        sl = pl.ds(cr * PCH, STRIP)
        rd_r = push(cbuf.at[sl, :], cbuf.at[sl, :],
                    pag_send_r.at[s_i], pag_recv_r.at[s_i], p_right)
        cl = lax.rem(p + 3 + s_i, 4)
        sl = pl.ds(cl * PCH + STRIP, STRIP)
        rd_l = push(cbuf.at[sl, :], cbuf.at[sl, :],
                    pag_send_l.at[s_i], pag_recv_l.at[s_i], p_left)
        rd_r.wait_recv()
        rd_l.wait_recv()

    out_ref[...] = cbuf[...].astype(f32)

    for rd in sends:
        rd.wait_send()


def kernel(x, Wq, K_ext, V_ext, Wo):
    x2 = x.reshape(SQ, SQ).astype(jnp.bfloat16)
    k2 = K_ext.reshape(SKV, D_LOC).astype(jnp.bfloat16)
    v2 = V_ext.reshape(SKV, D_LOC).astype(jnp.bfloat16)

    out = pl.pallas_call(
        _body,
        out_shape=jax.ShapeDtypeStruct((SQ, SQ), jnp.float32),
        in_specs=[
            pl.BlockSpec(memory_space=pltpu.VMEM),
            pl.BlockSpec(memory_space=pltpu.MemorySpace.HBM),
            pl.BlockSpec(memory_space=pltpu.VMEM),
            pl.BlockSpec(memory_space=pltpu.VMEM),
            pl.BlockSpec(memory_space=pltpu.MemorySpace.HBM),
        ],
        out_specs=pl.BlockSpec(memory_space=pltpu.VMEM),
        scratch_shapes=(
            [
                pltpu.VMEM((SQ, D_LOC), jnp.float32),
                pltpu.VMEM((D_LOC, SQ), jnp.float32),
                pltpu.VMEM((SQ, SQ), jnp.bfloat16),
                pltpu.VMEM((3, STRIP, SQ), jnp.bfloat16),
                pltpu.VMEM((3, STRIP, SQ), jnp.bfloat16),
                pltpu.VMEM((2, 64, SQ), jnp.bfloat16),
                pltpu.VMEM((2, 32, SQ), jnp.bfloat16),
                pltpu.SemaphoreType.DMA,
                pltpu.SemaphoreType.DMA,
            ]
            + [pltpu.SemaphoreType.DMA((3,))] * 4
            + [pltpu.SemaphoreType.DMA((8,))] * 2
            + [pltpu.SemaphoreType.DMA((3,))] * 4
        ),
        compiler_params=pltpu.CompilerParams(collective_id=0),
    )(x2, Wq, k2, v2, Wo)
    return out.reshape(1, SQ, SQ)


# device time: 70917 ns/iter; 2.4964x vs baseline; 1.0476x over previous
import os

import jax
import jax.numpy as jnp
from jax import lax
from jax.experimental import pallas as pl
from jax.experimental.pallas import tpu as pltpu

_KPHASE = os.environ.get("KPHASE", "full")

N_DEV = 16
SQ = 1024
SKV = 1024
H_LOC = 8
DH = 128
D_LOC = H_LOC * DH
SCALE = 0.08838834764831843

PCH = 256
STRIP = 128


def _body(x_ref, wq_hbm, k_ref, v_ref, wo_hbm, out_ref,
          wq_vmem, wo_vmem, cbuf, pbuf_r, pbuf_l, zbuf1, zbuf2,
          wq_sem, wo_sem,
          prs_send_r, prs_recv_r, prs_send_l, prs_recv_l,
          z_send, z_recv,
          pag_send_r, pag_recv_r, pag_send_l, pag_recv_l):
    my = lax.axis_index("i")
    z = my // 4
    p = lax.rem(my, 4)
    b0 = lax.rem(z, 2)
    b1 = z // 2
    p_right = my - p + lax.rem(p + 1, 4)
    p_left = my - p + lax.rem(p + 3, 4)
    z1p = my ^ 4
    z2p = my ^ 8

    wq_dma = pltpu.make_async_copy(
        wq_hbm.at[:, pl.ds(my * D_LOC, D_LOC)], wq_vmem, wq_sem)
    wq_dma.start()
    wo_dma = pltpu.make_async_copy(
        wo_hbm.at[pl.ds(my * D_LOC, D_LOC), :], wo_vmem, wo_sem)
    wo_dma.start()

    barrier_sem = pltpu.get_barrier_semaphore()
    for nbr in (p_left, p_right, z1p, z2p):
        pl.semaphore_signal(barrier_sem, inc=1, device_id=(nbr,),
                            device_id_type=pl.DeviceIdType.MESH)
    pl.semaphore_wait(barrier_sem, 4)

    f32 = jnp.float32
    bf16 = jnp.bfloat16
    _comm = _KPHASE != "compute"

    wq_dma.wait()
    wqb = wq_vmem[...].astype(bf16)
    wo_dma.wait()
    wob = wo_vmem[...].astype(bf16)

    def compute_strip(r0):
        xs = x_ref[pl.ds(r0, STRIP), :]
        q_s = jnp.dot(xs, wqb, preferred_element_type=f32)
        rows_b = (lax.broadcasted_iota(jnp.int32, (STRIP, SKV), 0) + r0) // 64
        cols_b = lax.broadcasted_iota(jnp.int32, (STRIP, SKV), 1) // 64
        mask = (rows_b == cols_b) | (cols_b == 0) | (
            lax.rem(rows_b + cols_b, 3) == 0)
        maskadd = jnp.where(mask, 0.0, -1e9).astype(f32)
        ctxs = []
        for h in range(H_LOC):
            hs = slice(h * DH, (h + 1) * DH)
            qh = q_s[:, hs].astype(bf16)
            s = lax.dot_general(qh, k_ref[:, hs], (((1,), (1,)), ((), ())),
                                preferred_element_type=f32)
            e = jnp.exp(s * SCALE + maskadd)
            recip = 1.0 / jnp.sum(e, axis=1, keepdims=True)
            ctx = jnp.dot(e.astype(bf16), v_ref[:, hs],
                          preferred_element_type=f32)
            ctxs.append(ctx * recip)
        ctx = jnp.concatenate(ctxs, axis=1).astype(bf16)
        cbuf[pl.ds(r0, STRIP), :] = jnp.dot(
            ctx, wob, preferred_element_type=f32).astype(bf16)

    sends = []

    def push(src, dst, ssem, rsem, dev):
        r = pltpu.make_async_remote_copy(
            src_ref=src, dst_ref=dst, send_sem=ssem, recv_sem=rsem,
            device_id=(dev,), device_id_type=pl.DeviceIdType.MESH)
        r.start()
        sends.append(r)
        return r

    tops = [p, lax.rem(p + 3, 4), lax.rem(p + 2, 4), lax.rem(p + 1, 4)]
    bots = [p, lax.rem(p + 1, 4), lax.rem(p + 2, 4), lax.rem(p + 3, 4)]
    compute_strip(tops[0] * PCH)
    compute_strip(bots[0] * PCH + STRIP)
    for s_i in range(3):
        if _comm:
            rd_r = push(cbuf.at[pl.ds(tops[s_i] * PCH, STRIP), :],
                        pbuf_r.at[s_i],
                        prs_send_r.at[s_i], prs_recv_r.at[s_i], p_right)
            rd_l = push(cbuf.at[pl.ds(bots[s_i] * PCH + STRIP, STRIP), :],
                        pbuf_l.at[s_i],
                        prs_send_l.at[s_i], prs_recv_l.at[s_i], p_left)
        compute_strip(tops[s_i + 1] * PCH)
        compute_strip(bots[s_i + 1] * PCH + STRIP)
        if _comm:
            rd_r.wait_recv()
            sl = pl.ds(tops[s_i + 1] * PCH, STRIP)
            cbuf[sl, :] = cbuf[sl, :] + pbuf_r[s_i]
            rd_l.wait_recv()
            sl = pl.ds(bots[s_i + 1] * PCH + STRIP, STRIP)
            cbuf[sl, :] = cbuf[sl, :] + pbuf_l[s_i]

    if not _comm:
        out_ref[...] = cbuf[...].astype(f32)
        return

    base_r = lax.rem(p + 1, 4) * PCH
    base_l = lax.rem(p + 3, 4) * PCH + STRIP
    bases = (base_r, base_l)

    rds = []
    for si, base in enumerate(bases):
        rds.append(push(cbuf.at[pl.ds(base + (1 - b0) * 64, 64), :],
                        zbuf1.at[si], z_send.at[si], z_recv.at[si], z1p))
    for si, base in enumerate(bases):
        rds[si].wait_recv()
        sl = pl.ds(base + b0 * 64, 64)
        cbuf[sl, :] = cbuf[sl, :] + zbuf1[si]
    q_keep = b0 * 64 + b1 * 32
    q_send = b0 * 64 + (1 - b1) * 32
    rds = []
    for si, base in enumerate(bases):
        rds.append(push(cbuf.at[pl.ds(base + q_send, 32), :],
                        zbuf2.at[si], z_send.at[2 + si], z_recv.at[2 + si], z2p))
    for si, base in enumerate(bases):
        rds[si].wait_recv()
        sl = pl.ds(base + q_keep, 32)
        cbuf[sl, :] = cbuf[sl, :] + zbuf2[si]
    rds = []
    for si, base in enumerate(bases):
        sl = pl.ds(base + q_keep, 32)
        rds.append(push(cbuf.at[sl, :], cbuf.at[sl, :],
                        z_send.at[4 + si], z_recv.at[4 + si], z2p))
    for rd in rds:
        rd.wait_recv()
    rds = []
    for si, base in enumerate(bases):
        sl = pl.ds(base + b0 * 64, 64)
        rds.append(push(cbuf.at[sl, :], cbuf.at[sl, :],
                        z_send.at[6 + si], z_recv.at[6 + si], z1p))
    for rd in rds:
        rd.wait_recv()

    sl = pl.ds(base_r, STRIP)
    out_ref[sl, :] = cbuf[sl, :].astype(f32)
    sl = pl.ds(base_l, STRIP)
    out_ref[sl, :] = cbuf[sl, :].astype(f32)

    for s_i in range(3):
        cr = lax.rem(p + 1 - s_i + 4, 4)
        sl = pl.ds(cr * PCH, STRIP)
        rd_r = push(cbuf.at[sl, :], cbuf.at[sl, :],
                    pag_send_r.at[s_i], pag_recv_r.at[s_i], p_right)
        cl = lax.rem(p + 3 + s_i, 4)
        sl = pl.ds(cl * PCH + STRIP, STRIP)
        rd_l = push(cbuf.at[sl, :], cbuf.at[sl, :],
                    pag_send_l.at[s_i], pag_recv_l.at[s_i], p_left)
        rd_r.wait_recv()
        sl = pl.ds(lax.rem(p - s_i + 4, 4) * PCH, STRIP)
        out_ref[sl, :] = cbuf[sl, :].astype(f32)
        rd_l.wait_recv()
        sl = pl.ds(lax.rem(p + s_i, 4) * PCH + STRIP, STRIP)
        out_ref[sl, :] = cbuf[sl, :].astype(f32)

    for rd in sends:
        rd.wait_send()


def kernel(x, Wq, K_ext, V_ext, Wo):
    x2 = x.reshape(SQ, SQ).astype(jnp.bfloat16)
    k2 = K_ext.reshape(SKV, D_LOC).astype(jnp.bfloat16)
    v2 = V_ext.reshape(SKV, D_LOC).astype(jnp.bfloat16)

    out = pl.pallas_call(
        _body,
        out_shape=jax.ShapeDtypeStruct((SQ, SQ), jnp.float32),
        in_specs=[
            pl.BlockSpec(memory_space=pltpu.VMEM),
            pl.BlockSpec(memory_space=pltpu.MemorySpace.HBM),
            pl.BlockSpec(memory_space=pltpu.VMEM),
            pl.BlockSpec(memory_space=pltpu.VMEM),
            pl.BlockSpec(memory_space=pltpu.MemorySpace.HBM),
        ],
        out_specs=pl.BlockSpec(memory_space=pltpu.VMEM),
        scratch_shapes=(
            [
                pltpu.VMEM((SQ, D_LOC), jnp.float32),
                pltpu.VMEM((D_LOC, SQ), jnp.float32),
                pltpu.VMEM((SQ, SQ), jnp.bfloat16),
                pltpu.VMEM((3, STRIP, SQ), jnp.bfloat16),
                pltpu.VMEM((3, STRIP, SQ), jnp.bfloat16),
                pltpu.VMEM((2, 64, SQ), jnp.bfloat16),
                pltpu.VMEM((2, 32, SQ), jnp.bfloat16),
                pltpu.SemaphoreType.DMA,
                pltpu.SemaphoreType.DMA,
            ]
            + [pltpu.SemaphoreType.DMA((3,))] * 4
            + [pltpu.SemaphoreType.DMA((8,))] * 2
            + [pltpu.SemaphoreType.DMA((3,))] * 4
        ),
        compiler_params=pltpu.CompilerParams(collective_id=0),
    )(x2, Wq, k2, v2, Wo)
    return out.reshape(1, SQ, SQ)


# device time: 70387 ns/iter; 2.5152x vs baseline; 1.0075x over previous
import os

import jax
import jax.numpy as jnp
from jax import lax
from jax.experimental import pallas as pl
from jax.experimental.pallas import tpu as pltpu

_KPHASE = os.environ.get("KPHASE", "full")

N_DEV = 16
SQ = 1024
SKV = 1024
H_LOC = 8
DH = 128
D_LOC = H_LOC * DH
SCALE = 0.08838834764831843

PCH = 256
STRIP = 128


def _body(x_ref, wq_hbm, k_ref, v_ref, wo_hbm, out_ref,
          wq_vmem, wo_vmem, cbuf, pbuf_r, pbuf_l, zbuf1, zbuf2,
          wq_sem, wo_sem,
          prs_send_r, prs_recv_r, prs_send_l, prs_recv_l,
          z_send, z_recv,
          pag_send_r, pag_recv_r, pag_send_l, pag_recv_l):
    my = lax.axis_index("i")
    z = my // 4
    p = lax.rem(my, 4)
    b0 = lax.rem(z, 2)
    b1 = z // 2
    p_right = my - p + lax.rem(p + 1, 4)
    p_left = my - p + lax.rem(p + 3, 4)
    z1p = my ^ 4
    z2p = my ^ 8

    wq_dma = pltpu.make_async_copy(
        wq_hbm.at[:, pl.ds(my * D_LOC, D_LOC)], wq_vmem, wq_sem)
    wq_dma.start()
    wo_dma = pltpu.make_async_copy(
        wo_hbm.at[pl.ds(my * D_LOC, D_LOC), :], wo_vmem, wo_sem)
    wo_dma.start()

    barrier_sem = pltpu.get_barrier_semaphore()
    for nbr in (p_left, p_right, z1p, z2p):
        pl.semaphore_signal(barrier_sem, inc=1, device_id=(nbr,),
                            device_id_type=pl.DeviceIdType.MESH)
    pl.semaphore_wait(barrier_sem, 4)

    f32 = jnp.float32
    bf16 = jnp.bfloat16
    _comm = _KPHASE != "compute"

    wq_dma.wait()
    wqb = wq_vmem[...].astype(bf16)
    wo_dma.wait()
    wob = wo_vmem[...].astype(bf16)

    def compute_chunk(c):
        r0 = c * PCH
        xs = x_ref[pl.ds(r0, PCH), :]
        q_s = jnp.dot(xs, wqb, preferred_element_type=f32)
        rows_b = (lax.broadcasted_iota(jnp.int32, (PCH, SKV), 0) + r0) // 64
        cols_b = lax.broadcasted_iota(jnp.int32, (PCH, SKV), 1) // 64
        mask = (rows_b == cols_b) | (cols_b == 0) | (
            lax.rem(rows_b + cols_b, 3) == 0)
        maskadd = jnp.where(mask, 0.0, -1e9).astype(f32)
        ctxs = []
        for h in range(H_LOC):
            hs = slice(h * DH, (h + 1) * DH)
            qh = q_s[:, hs].astype(bf16)
            s = lax.dot_general(qh, k_ref[:, hs], (((1,), (1,)), ((), ())),
                                preferred_element_type=f32)
            e = jnp.exp(s * SCALE + maskadd)
            recip = 1.0 / jnp.sum(e, axis=1, keepdims=True)
            ctx = jnp.dot(e.astype(bf16), v_ref[:, hs],
                          preferred_element_type=f32)
            ctxs.append(ctx * recip)
        ctx = jnp.concatenate(ctxs, axis=1).astype(bf16)
        cbuf[pl.ds(r0, PCH), :] = jnp.dot(
            ctx, wob, preferred_element_type=f32).astype(bf16)

    sends = []

    def push(src, dst, ssem, rsem, dev):
        r = pltpu.make_async_remote_copy(
            src_ref=src, dst_ref=dst, send_sem=ssem, recv_sem=rsem,
            device_id=(dev,), device_id_type=pl.DeviceIdType.MESH)
        r.start()
        sends.append(r)
        return r

    tops = [p, lax.rem(p + 3, 4), lax.rem(p + 2, 4), lax.rem(p + 1, 4)]
    bots = [p, lax.rem(p + 1, 4), lax.rem(p + 2, 4), lax.rem(p + 3, 4)]
    compute_chunk(p)
    for s_i in range(3):
        if _comm:
            rd_r = push(cbuf.at[pl.ds(tops[s_i] * PCH, STRIP), :],
                        pbuf_r.at[s_i],
                        prs_send_r.at[s_i], prs_recv_r.at[s_i], p_right)
            rd_l = push(cbuf.at[pl.ds(bots[s_i] * PCH + STRIP, STRIP), :],
                        pbuf_l.at[s_i],
                        prs_send_l.at[s_i], prs_recv_l.at[s_i], p_left)
        if s_i == 0:
            compute_chunk(lax.rem(p + 3, 4))
            compute_chunk(lax.rem(p + 1, 4))
        elif s_i == 1:
            compute_chunk(lax.rem(p + 2, 4))
        if _comm:
            rd_r.wait_recv()
            sl = pl.ds(tops[s_i + 1] * PCH, STRIP)
            cbuf[sl, :] = cbuf[sl, :] + pbuf_r[s_i]
            rd_l.wait_recv()
            sl = pl.ds(bots[s_i + 1] * PCH + STRIP, STRIP)
            cbuf[sl, :] = cbuf[sl, :] + pbuf_l[s_i]

    if not _comm:
        out_ref[...] = cbuf[...].astype(f32)
        return

    base_r = lax.rem(p + 1, 4) * PCH
    base_l = lax.rem(p + 3, 4) * PCH + STRIP
    bases = (base_r, base_l)

    rds = []
    for si, base in enumerate(bases):
        rds.append(push(cbuf.at[pl.ds(base + (1 - b0) * 64, 64), :],
                        zbuf1.at[si], z_send.at[si], z_recv.at[si], z1p))
    for si, base in enumerate(bases):
        rds[si].wait_recv()
        sl = pl.ds(base + b0 * 64, 64)
        cbuf[sl, :] = cbuf[sl, :] + zbuf1[si]
    q_keep = b0 * 64 + b1 * 32
    q_send = b0 * 64 + (1 - b1) * 32
    rds = []
    for si, base in enumerate(bases):
        rds.append(push(cbuf.at[pl.ds(base + q_send, 32), :],
                        zbuf2.at[si], z_send.at[2 + si], z_recv.at[2 + si], z2p))
    for si, base in enumerate(bases):
        rds[si].wait_recv()
        sl = pl.ds(base + q_keep, 32)
        cbuf[sl, :] = cbuf[sl, :] + zbuf2[si]
    rds = []
    for si, base in enumerate(bases):
        sl = pl.ds(base + q_keep, 32)
        rds.append(push(cbuf.at[sl, :], cbuf.at[sl, :],
                        z_send.at[4 + si], z_recv.at[4 + si], z2p))
    for rd in rds:
        rd.wait_recv()
    rds = []
    for si, base in enumerate(bases):
        sl = pl.ds(base + b0 * 64, 64)
        rds.append(push(cbuf.at[sl, :], cbuf.at[sl, :],
                        z_send.at[6 + si], z_recv.at[6 + si], z1p))
    for rd in rds:
        rd.wait_recv()

    sl = pl.ds(base_r, STRIP)
    out_ref[sl, :] = cbuf[sl, :].astype(f32)
    sl = pl.ds(base_l, STRIP)
    out_ref[sl, :] = cbuf[sl, :].astype(f32)

    for s_i in range(3):
        cr = lax.rem(p + 1 - s_i + 4, 4)
        sl = pl.ds(cr * PCH, STRIP)
        rd_r = push(cbuf.at[sl, :], cbuf.at[sl, :],
                    pag_send_r.at[s_i], pag_recv_r.at[s_i], p_right)
        cl = lax.rem(p + 3 + s_i, 4)
        sl = pl.ds(cl * PCH + STRIP, STRIP)
        rd_l = push(cbuf.at[sl, :], cbuf.at[sl, :],
                    pag_send_l.at[s_i], pag_recv_l.at[s_i], p_left)
        rd_r.wait_recv()
        sl = pl.ds(lax.rem(p - s_i + 4, 4) * PCH, STRIP)
        out_ref[sl, :] = cbuf[sl, :].astype(f32)
        rd_l.wait_recv()
        sl = pl.ds(lax.rem(p + s_i, 4) * PCH + STRIP, STRIP)
        out_ref[sl, :] = cbuf[sl, :].astype(f32)

    for rd in sends:
        rd.wait_send()


def kernel(x, Wq, K_ext, V_ext, Wo):
    x2 = x.reshape(SQ, SQ).astype(jnp.bfloat16)
    k2 = K_ext.reshape(SKV, D_LOC).astype(jnp.bfloat16)
    v2 = V_ext.reshape(SKV, D_LOC).astype(jnp.bfloat16)

    out = pl.pallas_call(
        _body,
        out_shape=jax.ShapeDtypeStruct((SQ, SQ), jnp.float32),
        in_specs=[
            pl.BlockSpec(memory_space=pltpu.VMEM),
            pl.BlockSpec(memory_space=pltpu.MemorySpace.HBM),
            pl.BlockSpec(memory_space=pltpu.VMEM),
            pl.BlockSpec(memory_space=pltpu.VMEM),
            pl.BlockSpec(memory_space=pltpu.MemorySpace.HBM),
        ],
        out_specs=pl.BlockSpec(memory_space=pltpu.VMEM),
        scratch_shapes=(
            [
                pltpu.VMEM((SQ, D_LOC), jnp.float32),
                pltpu.VMEM((D_LOC, SQ), jnp.float32),
                pltpu.VMEM((SQ, SQ), jnp.bfloat16),
                pltpu.VMEM((3, STRIP, SQ), jnp.bfloat16),
                pltpu.VMEM((3, STRIP, SQ), jnp.bfloat16),
                pltpu.VMEM((2, 64, SQ), jnp.bfloat16),
                pltpu.VMEM((2, 32, SQ), jnp.bfloat16),
                pltpu.SemaphoreType.DMA,
                pltpu.SemaphoreType.DMA,
            ]
            + [pltpu.SemaphoreType.DMA((3,))] * 4
            + [pltpu.SemaphoreType.DMA((8,))] * 2
            + [pltpu.SemaphoreType.DMA((3,))] * 4
        ),
        compiler_params=pltpu.CompilerParams(collective_id=0),
    )(x2, Wq, k2, v2, Wo)
    return out.reshape(1, SQ, SQ)


# device time: 64372 ns/iter; 2.7502x vs baseline; 1.0934x over previous
import os

import jax
import jax.numpy as jnp
from jax import lax
from jax.experimental import pallas as pl
from jax.experimental.pallas import tpu as pltpu

_KPHASE = os.environ.get("KPHASE", "full")

N_DEV = 16
SQ = 1024
SKV = 1024
H_LOC = 8
DH = 128
D_LOC = H_LOC * DH
SCALE = 0.08838834764831843

PCH = 256
STRIP = 128


def _body(x_ref, wq_hbm, k_ref, v_ref, wo_hbm, out_ref,
          wq_vmem, wo_vmem, cbuf, pbuf_r, pbuf_l, zbuf1, zbuf2,
          wq_sem, wo_sem,
          prs_send_r, prs_recv_r, prs_send_l, prs_recv_l,
          z_send, z_recv,
          pag_send_r, pag_recv_r, pag_send_l, pag_recv_l):
    my = lax.axis_index("i")
    z = my // 4
    p = lax.rem(my, 4)
    b0 = lax.rem(z, 2)
    b1 = z // 2
    p_right = my - p + lax.rem(p + 1, 4)
    p_left = my - p + lax.rem(p + 3, 4)
    z1p = my ^ 4
    z2p = my ^ 8

    wq_dma = pltpu.make_async_copy(
        wq_hbm.at[:, pl.ds(my * D_LOC, D_LOC)], wq_vmem, wq_sem)
    wq_dma.start()
    wo_dma = pltpu.make_async_copy(
        wo_hbm.at[pl.ds(my * D_LOC, D_LOC), :], wo_vmem, wo_sem)
    wo_dma.start()

    barrier_sem = pltpu.get_barrier_semaphore()
    for nbr in (p_left, p_right, z1p, z2p):
        pl.semaphore_signal(barrier_sem, inc=1, device_id=(nbr,),
                            device_id_type=pl.DeviceIdType.MESH)
    pl.semaphore_wait(barrier_sem, 4)

    f32 = jnp.float32
    bf16 = jnp.bfloat16
    _comm = _KPHASE != "compute"

    wq_dma.wait()
    wqb = (wq_vmem[...] * SCALE).astype(bf16)
    wo_dma.wait()
    wob = wo_vmem[...].astype(bf16)

    def compute_chunk(c):
        r0 = c * PCH
        xs = x_ref[pl.ds(r0, PCH), :]
        q_s = jnp.dot(xs, wqb, preferred_element_type=f32)
        rows_b = (lax.broadcasted_iota(jnp.int32, (PCH, SKV), 0) + r0) // 64
        cols_b = lax.broadcasted_iota(jnp.int32, (PCH, SKV), 1) // 64
        mask = (rows_b == cols_b) | (cols_b == 0) | (
            lax.rem(rows_b + cols_b, 3) == 0)
        maskadd = jnp.where(mask, 0.0, -1e9).astype(f32)
        ctxs = []
        for h in range(H_LOC):
            hs = slice(h * DH, (h + 1) * DH)
            qh = q_s[:, hs].astype(bf16)
            s = lax.dot_general(qh, k_ref[:, hs], (((1,), (1,)), ((), ())),
                                preferred_element_type=f32)
            e = jnp.exp(s + maskadd)
            recip = 1.0 / jnp.sum(e, axis=1, keepdims=True)
            ctx = jnp.dot(e.astype(bf16), v_ref[:, hs],
                          preferred_element_type=f32)
            ctxs.append(ctx * recip)
        ctx = jnp.concatenate(ctxs, axis=1).astype(bf16)
        cbuf[pl.ds(r0, PCH), :] = jnp.dot(
            ctx, wob, preferred_element_type=f32).astype(bf16)

    sends = []

    def push(src, dst, ssem, rsem, dev):
        r = pltpu.make_async_remote_copy(
            src_ref=src, dst_ref=dst, send_sem=ssem, recv_sem=rsem,
            device_id=(dev,), device_id_type=pl.DeviceIdType.MESH)
        r.start()
        sends.append(r)
        return r

    tops = [p, lax.rem(p + 3, 4), lax.rem(p + 2, 4), lax.rem(p + 1, 4)]
    bots = [p, lax.rem(p + 1, 4), lax.rem(p + 2, 4), lax.rem(p + 3, 4)]
    compute_chunk(p)
    for s_i in range(3):
        if _comm:
            rd_r = push(cbuf.at[pl.ds(tops[s_i] * PCH, STRIP), :],
                        pbuf_r.at[s_i],
                        prs_send_r.at[s_i], prs_recv_r.at[s_i], p_right)
            rd_l = push(cbuf.at[pl.ds(bots[s_i] * PCH + STRIP, STRIP), :],
                        pbuf_l.at[s_i],
                        prs_send_l.at[s_i], prs_recv_l.at[s_i], p_left)
        if s_i == 0:
            compute_chunk(lax.rem(p + 3, 4))
            compute_chunk(lax.rem(p + 1, 4))
        elif s_i == 1:
            compute_chunk(lax.rem(p + 2, 4))
        if _comm:
            rd_r.wait_recv()
            sl = pl.ds(tops[s_i + 1] * PCH, STRIP)
            cbuf[sl, :] = cbuf[sl, :] + pbuf_r[s_i]
            rd_l.wait_recv()
            sl = pl.ds(bots[s_i + 1] * PCH + STRIP, STRIP)
            cbuf[sl, :] = cbuf[sl, :] + pbuf_l[s_i]

    if not _comm:
        out_ref[...] = cbuf[...].astype(f32)
        return

    if _KPHASE == "rs":
        out_ref[...] = cbuf[...].astype(f32)
        for rd in sends:
            rd.wait_send()
        return

    base_r = lax.rem(p + 1, 4) * PCH
    base_l = lax.rem(p + 3, 4) * PCH + STRIP
    bases = (base_r, base_l)

    rds = []
    for si, base in enumerate(bases):
        rds.append(push(cbuf.at[pl.ds(base + (1 - b0) * 64, 64), :],
                        zbuf1.at[si], z_send.at[si], z_recv.at[si], z1p))
    for si, base in enumerate(bases):
        rds[si].wait_recv()
        sl = pl.ds(base + b0 * 64, 64)
        cbuf[sl, :] = cbuf[sl, :] + zbuf1[si]
    q_keep = b0 * 64 + b1 * 32
    q_send = b0 * 64 + (1 - b1) * 32
    rds = []
    for si, base in enumerate(bases):
        rds.append(push(cbuf.at[pl.ds(base + q_send, 32), :],
                        zbuf2.at[si], z_send.at[2 + si], z_recv.at[2 + si], z2p))
    for si, base in enumerate(bases):
        rds[si].wait_recv()
        sl = pl.ds(base + q_keep, 32)
        cbuf[sl, :] = cbuf[sl, :] + zbuf2[si]
    rds = []
    for si, base in enumerate(bases):
        sl = pl.ds(base + q_keep, 32)
        rds.append(push(cbuf.at[sl, :], cbuf.at[sl, :],
                        z_send.at[4 + si], z_recv.at[4 + si], z2p))
    for rd in rds:
        rd.wait_recv()
    if _KPHASE == "rsz":
        out_ref[...] = cbuf[...].astype(f32)
        for rd in sends:
            rd.wait_send()
        return

    hoff0 = b0 * 64
    hoff1 = (1 - b0) * 64
    ag_tops = [lax.rem(p + 1 - s + 4, 4) for s in range(3)]
    ag_bots = [lax.rem(p + 3 + s, 4) for s in range(3)]
    rc_tops = [lax.rem(p - s + 4, 4) for s in range(3)]
    rc_bots = [lax.rem(p + s, 4) for s in range(3)]

    def ag_send(h, s, hoff):
        i = h * 3 + s
        sl_t = pl.ds(ag_tops[s] * PCH + hoff, 64)
        rr = push(cbuf.at[sl_t, :], cbuf.at[sl_t, :],
                  pag_send_r.at[i], pag_recv_r.at[i], p_right)
        sl_b = pl.ds(ag_bots[s] * PCH + STRIP + hoff, 64)
        rl = push(cbuf.at[sl_b, :], cbuf.at[sl_b, :],
                  pag_send_l.at[i], pag_recv_l.at[i], p_left)
        return rr, rl

    def ag_wait(pair, s, hoff):
        rr, rl = pair
        rr.wait_recv()
        sl = pl.ds(rc_tops[s] * PCH + hoff, 64)
        out_ref[sl, :] = cbuf[sl, :].astype(f32)
        rl.wait_recv()
        sl = pl.ds(rc_bots[s] * PCH + STRIP + hoff, 64)
        out_ref[sl, :] = cbuf[sl, :].astype(f32)

    a0 = ag_send(0, 0, hoff0)
    rds = []
    for si, base in enumerate(bases):
        sl = pl.ds(base + hoff0, 64)
        rds.append(push(cbuf.at[sl, :], cbuf.at[sl, :],
                        z_send.at[6 + si], z_recv.at[6 + si], z1p))
    sl = pl.ds(base_r + hoff0, 64)
    out_ref[sl, :] = cbuf[sl, :].astype(f32)
    sl = pl.ds(base_l + hoff0, 64)
    out_ref[sl, :] = cbuf[sl, :].astype(f32)
    for rd in rds:
        rd.wait_recv()
    c0 = ag_send(1, 0, hoff1)
    sl = pl.ds(base_r + hoff1, 64)
    out_ref[sl, :] = cbuf[sl, :].astype(f32)
    sl = pl.ds(base_l + hoff1, 64)
    out_ref[sl, :] = cbuf[sl, :].astype(f32)
    ag_wait(a0, 0, hoff0)
    a1 = ag_send(0, 1, hoff0)
    ag_wait(c0, 0, hoff1)
    c1 = ag_send(1, 1, hoff1)
    ag_wait(a1, 1, hoff0)
    a2 = ag_send(0, 2, hoff0)
    ag_wait(c1, 1, hoff1)
    c2 = ag_send(1, 2, hoff1)
    ag_wait(a2, 2, hoff0)
    ag_wait(c2, 2, hoff1)

    for rd in sends:
        rd.wait_send()


def kernel(x, Wq, K_ext, V_ext, Wo):
    x2 = x.reshape(SQ, SQ).astype(jnp.bfloat16)
    k2 = K_ext.reshape(SKV, D_LOC).astype(jnp.bfloat16)
    v2 = V_ext.reshape(SKV, D_LOC).astype(jnp.bfloat16)

    out = pl.pallas_call(
        _body,
        out_shape=jax.ShapeDtypeStruct((SQ, SQ), jnp.float32),
        in_specs=[
            pl.BlockSpec(memory_space=pltpu.VMEM),
            pl.BlockSpec(memory_space=pltpu.MemorySpace.HBM),
            pl.BlockSpec(memory_space=pltpu.VMEM),
            pl.BlockSpec(memory_space=pltpu.VMEM),
            pl.BlockSpec(memory_space=pltpu.MemorySpace.HBM),
        ],
        out_specs=pl.BlockSpec(memory_space=pltpu.VMEM),
        scratch_shapes=(
            [
                pltpu.VMEM((SQ, D_LOC), jnp.float32),
                pltpu.VMEM((D_LOC, SQ), jnp.float32),
                pltpu.VMEM((SQ, SQ), jnp.bfloat16),
                pltpu.VMEM((3, STRIP, SQ), jnp.bfloat16),
                pltpu.VMEM((3, STRIP, SQ), jnp.bfloat16),
                pltpu.VMEM((2, 64, SQ), jnp.bfloat16),
                pltpu.VMEM((2, 32, SQ), jnp.bfloat16),
                pltpu.SemaphoreType.DMA,
                pltpu.SemaphoreType.DMA,
            ]
            + [pltpu.SemaphoreType.DMA((3,))] * 4
            + [pltpu.SemaphoreType.DMA((8,))] * 2
            + [pltpu.SemaphoreType.DMA((6,))] * 4
        ),
        compiler_params=pltpu.CompilerParams(collective_id=0),
    )(x2, Wq, k2, v2, Wo)
    return out.reshape(1, SQ, SQ)


# device time: 62955 ns/iter; 2.8121x vs baseline; 1.0225x over previous
import os

import jax
import jax.numpy as jnp
from jax import lax
from jax.experimental import pallas as pl
from jax.experimental.pallas import tpu as pltpu

_KPHASE = os.environ.get("KPHASE", "full")

N_DEV = 16
SQ = 1024
SKV = 1024
H_LOC = 8
DH = 128
D_LOC = H_LOC * DH
SCALE = 0.08838834764831843

PCH = 256
STRIP = 128


def _body(x_ref, wq_hbm, k_ref, v_ref, wo_hbm, out_ref,
          wq_vmem, wo_vmem, cbuf, mask_ref, pbuf_r, pbuf_l, zbuf1, zbuf2,
          wq_sem, wo_sem,
          prs_send_r, prs_recv_r, prs_send_l, prs_recv_l,
          z_send, z_recv,
          pag_send_r, pag_recv_r, pag_send_l, pag_recv_l):
    my = lax.axis_index("i")
    z = my // 4
    p = lax.rem(my, 4)
    b0 = lax.rem(z, 2)
    b1 = z // 2
    p_right = my - p + lax.rem(p + 1, 4)
    p_left = my - p + lax.rem(p + 3, 4)
    z1p = my ^ 4
    z2p = my ^ 8

    wq_dma = pltpu.make_async_copy(
        wq_hbm.at[:, pl.ds(my * D_LOC, D_LOC)], wq_vmem, wq_sem)
    wq_dma.start()
    wo_dma = pltpu.make_async_copy(
        wo_hbm.at[pl.ds(my * D_LOC, D_LOC), :], wo_vmem, wo_sem)
    wo_dma.start()

    barrier_sem = pltpu.get_barrier_semaphore()
    for nbr in (p_left, p_right, z1p, z2p):
        pl.semaphore_signal(barrier_sem, inc=1, device_id=(nbr,),
                            device_id_type=pl.DeviceIdType.MESH)
    pl.semaphore_wait(barrier_sem, 4)

    f32 = jnp.float32
    bf16 = jnp.bfloat16
    _comm = _KPHASE != "compute"

    rows_b = lax.broadcasted_iota(jnp.int32, (SQ, SKV), 0) // 64
    cols_b = lax.broadcasted_iota(jnp.int32, (SQ, SKV), 1) // 64
    mask = (rows_b == cols_b) | (cols_b == 0) | (
        lax.rem(rows_b + cols_b, 3) == 0)
    mask_ref[...] = jnp.where(mask, 0.0, -1e9).astype(f32)

    wq_dma.wait()
    wqb = (wq_vmem[...] * SCALE).astype(bf16)
    wo_dma.wait()
    wob = wo_vmem[...].astype(bf16)

    def compute_chunk(c):
        r0 = c * PCH
        xs = x_ref[pl.ds(r0, PCH), :]
        q_s = jnp.dot(xs, wqb, preferred_element_type=f32)
        maskadd = mask_ref[pl.ds(r0, PCH), :]
        ctxs = []
        for h in range(H_LOC):
            hs = slice(h * DH, (h + 1) * DH)
            qh = q_s[:, hs].astype(bf16)
            s = lax.dot_general(qh, k_ref[:, hs], (((1,), (1,)), ((), ())),
                                preferred_element_type=f32)
            e = jnp.exp(s + maskadd)
            recip = 1.0 / jnp.sum(e, axis=1, keepdims=True)
            ctx = jnp.dot(e.astype(bf16), v_ref[:, hs],
                          preferred_element_type=f32)
            ctxs.append(ctx * recip)
        ctx = jnp.concatenate(ctxs, axis=1).astype(bf16)
        cbuf[pl.ds(r0, PCH), :] = jnp.dot(
            ctx, wob, preferred_element_type=f32).astype(bf16)

    sends = []

    def push(src, dst, ssem, rsem, dev):
        r = pltpu.make_async_remote_copy(
            src_ref=src, dst_ref=dst, send_sem=ssem, recv_sem=rsem,
            device_id=(dev,), device_id_type=pl.DeviceIdType.MESH)
        r.start()
        sends.append(r)
        return r

    tops = [p, lax.rem(p + 3, 4), lax.rem(p + 2, 4), lax.rem(p + 1, 4)]
    bots = [p, lax.rem(p + 1, 4), lax.rem(p + 2, 4), lax.rem(p + 3, 4)]
    compute_chunk(p)
    for s_i in range(3):
        if _comm:
            rd_r = push(cbuf.at[pl.ds(tops[s_i] * PCH, STRIP), :],
                        pbuf_r.at[s_i],
                        prs_send_r.at[s_i], prs_recv_r.at[s_i], p_right)
            rd_l = push(cbuf.at[pl.ds(bots[s_i] * PCH + STRIP, STRIP), :],
                        pbuf_l.at[s_i],
                        prs_send_l.at[s_i], prs_recv_l.at[s_i], p_left)
        if s_i == 0:
            compute_chunk(lax.rem(p + 3, 4))
            compute_chunk(lax.rem(p + 1, 4))
        elif s_i == 1:
            compute_chunk(lax.rem(p + 2, 4))
        if _comm:
            rd_r.wait_recv()
            sl = pl.ds(tops[s_i + 1] * PCH, STRIP)
            cbuf[sl, :] = cbuf[sl, :] + pbuf_r[s_i]
            rd_l.wait_recv()
            sl = pl.ds(bots[s_i + 1] * PCH + STRIP, STRIP)
            cbuf[sl, :] = cbuf[sl, :] + pbuf_l[s_i]

    if not _comm:
        out_ref[...] = cbuf[...].astype(f32)
        return

    if _KPHASE == "rs":
        out_ref[...] = cbuf[...].astype(f32)
        for rd in sends:
            rd.wait_send()
        return

    base_r = lax.rem(p + 1, 4) * PCH
    base_l = lax.rem(p + 3, 4) * PCH + STRIP
    bases = (base_r, base_l)

    rds = []
    for si, base in enumerate(bases):
        rds.append(push(cbuf.at[pl.ds(base + (1 - b0) * 64, 64), :],
                        zbuf1.at[si], z_send.at[si], z_recv.at[si], z1p))
    for si, base in enumerate(bases):
        rds[si].wait_recv()
        sl = pl.ds(base + b0 * 64, 64)
        cbuf[sl, :] = cbuf[sl, :] + zbuf1[si]
    q_keep = b0 * 64 + b1 * 32
    q_send = b0 * 64 + (1 - b1) * 32
    rds = []
    for si, base in enumerate(bases):
        rds.append(push(cbuf.at[pl.ds(base + q_send, 32), :],
                        zbuf2.at[si], z_send.at[2 + si], z_recv.at[2 + si], z2p))
    for si, base in enumerate(bases):
        rds[si].wait_recv()
        sl = pl.ds(base + q_keep, 32)
        cbuf[sl, :] = cbuf[sl, :] + zbuf2[si]
    rds = []
    for si, base in enumerate(bases):
        sl = pl.ds(base + q_keep, 32)
        rds.append(push(cbuf.at[sl, :], cbuf.at[sl, :],
                        z_send.at[4 + si], z_recv.at[4 + si], z2p))
    for rd in rds:
        rd.wait_recv()
    if _KPHASE == "rsz":
        out_ref[...] = cbuf[...].astype(f32)
        for rd in sends:
            rd.wait_send()
        return

    hoff0 = b0 * 64
    hoff1 = (1 - b0) * 64
    ag_tops = [lax.rem(p + 1 - s + 4, 4) for s in range(3)]
    ag_bots = [lax.rem(p + 3 + s, 4) for s in range(3)]
    rc_tops = [lax.rem(p - s + 4, 4) for s in range(3)]
    rc_bots = [lax.rem(p + s, 4) for s in range(3)]

    def ag_send(h, s, hoff):
        i = h * 3 + s
        sl_t = pl.ds(ag_tops[s] * PCH + hoff, 64)
        rr = push(cbuf.at[sl_t, :], cbuf.at[sl_t, :],
                  pag_send_r.at[i], pag_recv_r.at[i], p_right)
        sl_b = pl.ds(ag_bots[s] * PCH + STRIP + hoff, 64)
        rl = push(cbuf.at[sl_b, :], cbuf.at[sl_b, :],
                  pag_send_l.at[i], pag_recv_l.at[i], p_left)
        return rr, rl

    def ag_wait(pair, s, hoff):
        rr, rl = pair
        rr.wait_recv()
        sl = pl.ds(rc_tops[s] * PCH + hoff, 64)
        out_ref[sl, :] = cbuf[sl, :].astype(f32)
        rl.wait_recv()
        sl = pl.ds(rc_bots[s] * PCH + STRIP + hoff, 64)
        out_ref[sl, :] = cbuf[sl, :].astype(f32)

    a0 = ag_send(0, 0, hoff0)
    rds = []
    for si, base in enumerate(bases):
        sl = pl.ds(base + hoff0, 64)
        rds.append(push(cbuf.at[sl, :], cbuf.at[sl, :],
                        z_send.at[6 + si], z_recv.at[6 + si], z1p))
    sl = pl.ds(base_r + hoff0, 64)
    out_ref[sl, :] = cbuf[sl, :].astype(f32)
    sl = pl.ds(base_l + hoff0, 64)
    out_ref[sl, :] = cbuf[sl, :].astype(f32)
    for rd in rds:
        rd.wait_recv()
    c0 = ag_send(1, 0, hoff1)
    sl = pl.ds(base_r + hoff1, 64)
    out_ref[sl, :] = cbuf[sl, :].astype(f32)
    sl = pl.ds(base_l + hoff1, 64)
    out_ref[sl, :] = cbuf[sl, :].astype(f32)
    ag_wait(a0, 0, hoff0)
    a1 = ag_send(0, 1, hoff0)
    ag_wait(c0, 0, hoff1)
    c1 = ag_send(1, 1, hoff1)
    ag_wait(a1, 1, hoff0)
    a2 = ag_send(0, 2, hoff0)
    ag_wait(c1, 1, hoff1)
    c2 = ag_send(1, 2, hoff1)
    ag_wait(a2, 2, hoff0)
    ag_wait(c2, 2, hoff1)

    for rd in sends:
        rd.wait_send()


def kernel(x, Wq, K_ext, V_ext, Wo):
    x2 = x.reshape(SQ, SQ).astype(jnp.bfloat16)
    k2 = K_ext.reshape(SKV, D_LOC).astype(jnp.bfloat16)
    v2 = V_ext.reshape(SKV, D_LOC).astype(jnp.bfloat16)

    out = pl.pallas_call(
        _body,
        out_shape=jax.ShapeDtypeStruct((SQ, SQ), jnp.float32),
        in_specs=[
            pl.BlockSpec(memory_space=pltpu.VMEM),
            pl.BlockSpec(memory_space=pltpu.MemorySpace.HBM),
            pl.BlockSpec(memory_space=pltpu.VMEM),
            pl.BlockSpec(memory_space=pltpu.VMEM),
            pl.BlockSpec(memory_space=pltpu.MemorySpace.HBM),
        ],
        out_specs=pl.BlockSpec(memory_space=pltpu.VMEM),
        scratch_shapes=(
            [
                pltpu.VMEM((SQ, D_LOC), jnp.float32),
                pltpu.VMEM((D_LOC, SQ), jnp.float32),
                pltpu.VMEM((SQ, SQ), jnp.bfloat16),
                pltpu.VMEM((SQ, SKV), jnp.float32),
                pltpu.VMEM((3, STRIP, SQ), jnp.bfloat16),
                pltpu.VMEM((3, STRIP, SQ), jnp.bfloat16),
                pltpu.VMEM((2, 64, SQ), jnp.bfloat16),
                pltpu.VMEM((2, 32, SQ), jnp.bfloat16),
                pltpu.SemaphoreType.DMA,
                pltpu.SemaphoreType.DMA,
            ]
            + [pltpu.SemaphoreType.DMA((3,))] * 4
            + [pltpu.SemaphoreType.DMA((8,))] * 2
            + [pltpu.SemaphoreType.DMA((6,))] * 4
        ),
        compiler_params=pltpu.CompilerParams(collective_id=0),
    )(x2, Wq, k2, v2, Wo)
    return out.reshape(1, SQ, SQ)


# device time: 62126 ns/iter; 2.8496x vs baseline; 1.0133x over previous
import os

import jax
import jax.numpy as jnp
from jax import lax
from jax.experimental import pallas as pl
from jax.experimental.pallas import tpu as pltpu

_KPHASE = os.environ.get("KPHASE", "full")

N_DEV = 16
SQ = 1024
SKV = 1024
H_LOC = 8
DH = 128
D_LOC = H_LOC * DH
SCALE = 0.08838834764831843

PCH = 256
STRIP = 128


def _body(x_ref, wq_hbm, k_ref, v_ref, wo_hbm, out_ref,
          wq_vmem, wo_vmem, cbuf, mask_ref, pbuf_r, pbuf_l, zbuf1, zbuf2,
          wq_sem, wo_sem,
          prs_send_r, prs_recv_r, prs_send_l, prs_recv_l,
          z_send, z_recv,
          pag_send_r, pag_recv_r, pag_send_l, pag_recv_l):
    my = lax.axis_index("i")
    z = my // 4
    p = lax.rem(my, 4)
    b0 = lax.rem(z, 2)
    b1 = z // 2
    p_right = my - p + lax.rem(p + 1, 4)
    p_left = my - p + lax.rem(p + 3, 4)
    z1p = my ^ 4
    z2p = my ^ 8

    wq_dma = pltpu.make_async_copy(
        wq_hbm.at[:, pl.ds(my * D_LOC, D_LOC)], wq_vmem, wq_sem)
    wq_dma.start()
    wo_dma = pltpu.make_async_copy(
        wo_hbm.at[pl.ds(my * D_LOC, D_LOC), :], wo_vmem, wo_sem)
    wo_dma.start()

    barrier_sem = pltpu.get_barrier_semaphore()
    for nbr in (p_left, p_right, z1p, z2p):
        pl.semaphore_signal(barrier_sem, inc=1, device_id=(nbr,),
                            device_id_type=pl.DeviceIdType.MESH)
    pl.semaphore_wait(barrier_sem, 4)

    f32 = jnp.float32
    bf16 = jnp.bfloat16
    _comm = _KPHASE != "compute"

    rows_b = lax.broadcasted_iota(jnp.int32, (SQ, SKV), 0) // 64
    cols_b = lax.broadcasted_iota(jnp.int32, (SQ, SKV), 1) // 64
    mask = (rows_b == cols_b) | (cols_b == 0) | (
        lax.rem(rows_b + cols_b, 3) == 0)
    mask_ref[...] = jnp.where(mask, 0.0, -1e9).astype(bf16)
    kb = k_ref[...].astype(bf16)
    vb = v_ref[...].astype(bf16)

    wq_dma.wait()
    wqb = (wq_vmem[...] * SCALE).astype(bf16)
    wo_dma.wait()
    wob = wo_vmem[...].astype(bf16)

    def compute_chunk(c):
        r0 = c * PCH
        xs = x_ref[pl.ds(r0, PCH), :].astype(bf16)
        q_s = jnp.dot(xs, wqb, preferred_element_type=f32)
        maskadd = mask_ref[pl.ds(r0, PCH), :]
        ctxs = []
        for h in range(H_LOC):
            hs = slice(h * DH, (h + 1) * DH)
            qh = q_s[:, hs].astype(bf16)
            s = lax.dot_general(qh, kb[:, hs], (((1,), (1,)), ((), ())),
                                preferred_element_type=f32)
            e = jnp.exp(s + maskadd)
            recip = 1.0 / jnp.sum(e, axis=1, keepdims=True)
            ctx = jnp.dot(e.astype(bf16), vb[:, hs],
                          preferred_element_type=f32)
            ctxs.append(ctx * recip)
        ctx = jnp.concatenate(ctxs, axis=1).astype(bf16)
        cbuf[pl.ds(r0, PCH), :] = jnp.dot(
            ctx, wob, preferred_element_type=f32).astype(bf16)

    sends = []

    def push(src, dst, ssem, rsem, dev):
        r = pltpu.make_async_remote_copy(
            src_ref=src, dst_ref=dst, send_sem=ssem, recv_sem=rsem,
            device_id=(dev,), device_id_type=pl.DeviceIdType.MESH)
        r.start()
        sends.append(r)
        return r

    tops = [p, lax.rem(p + 3, 4), lax.rem(p + 2, 4), lax.rem(p + 1, 4)]
    bots = [p, lax.rem(p + 1, 4), lax.rem(p + 2, 4), lax.rem(p + 3, 4)]
    compute_chunk(p)
    for s_i in range(3):
        if _comm:
            rd_r = push(cbuf.at[pl.ds(tops[s_i] * PCH, STRIP), :],
                        pbuf_r.at[s_i],
                        prs_send_r.at[s_i], prs_recv_r.at[s_i], p_right)
            rd_l = push(cbuf.at[pl.ds(bots[s_i] * PCH + STRIP, STRIP), :],
                        pbuf_l.at[s_i],
                        prs_send_l.at[s_i], prs_recv_l.at[s_i], p_left)
        if s_i == 0:
            compute_chunk(lax.rem(p + 3, 4))
            compute_chunk(lax.rem(p + 1, 4))
        elif s_i == 1:
            compute_chunk(lax.rem(p + 2, 4))
        if _comm:
            rd_r.wait_recv()
            sl = pl.ds(tops[s_i + 1] * PCH, STRIP)
            cbuf[sl, :] = cbuf[sl, :] + pbuf_r[s_i]
            rd_l.wait_recv()
            sl = pl.ds(bots[s_i + 1] * PCH + STRIP, STRIP)
            cbuf[sl, :] = cbuf[sl, :] + pbuf_l[s_i]

    if not _comm:
        out_ref[...] = cbuf[...].astype(f32)
        return

    if _KPHASE == "rs":
        out_ref[...] = cbuf[...].astype(f32)
        for rd in sends:
            rd.wait_send()
        return

    base_r = lax.rem(p + 1, 4) * PCH
    base_l = lax.rem(p + 3, 4) * PCH + STRIP
    bases = (base_r, base_l)

    rds = []
    for si, base in enumerate(bases):
        rds.append(push(cbuf.at[pl.ds(base + (1 - b0) * 64, 64), :],
                        zbuf1.at[si], z_send.at[si], z_recv.at[si], z1p))
    for si, base in enumerate(bases):
        rds[si].wait_recv()
        sl = pl.ds(base + b0 * 64, 64)
        cbuf[sl, :] = cbuf[sl, :] + zbuf1[si]
    q_keep = b0 * 64 + b1 * 32
    q_send = b0 * 64 + (1 - b1) * 32
    rds = []
    for si, base in enumerate(bases):
        rds.append(push(cbuf.at[pl.ds(base + q_send, 32), :],
                        zbuf2.at[si], z_send.at[2 + si], z_recv.at[2 + si], z2p))
    for si, base in enumerate(bases):
        rds[si].wait_recv()
        sl = pl.ds(base + q_keep, 32)
        cbuf[sl, :] = cbuf[sl, :] + zbuf2[si]
    rds = []
    for si, base in enumerate(bases):
        sl = pl.ds(base + q_keep, 32)
        rds.append(push(cbuf.at[sl, :], cbuf.at[sl, :],
                        z_send.at[4 + si], z_recv.at[4 + si], z2p))
    for rd in rds:
        rd.wait_recv()
    if _KPHASE == "rsz":
        out_ref[...] = cbuf[...].astype(f32)
        for rd in sends:
            rd.wait_send()
        return

    hoff0 = b0 * 64
    hoff1 = (1 - b0) * 64
    ag_tops = [lax.rem(p + 1 - s + 4, 4) for s in range(3)]
    ag_bots = [lax.rem(p + 3 + s, 4) for s in range(3)]
    rc_tops = [lax.rem(p - s + 4, 4) for s in range(3)]
    rc_bots = [lax.rem(p + s, 4) for s in range(3)]

    def ag_send(h, s, hoff):
        i = h * 3 + s
        sl_t = pl.ds(ag_tops[s] * PCH + hoff, 64)
        rr = push(cbuf.at[sl_t, :], cbuf.at[sl_t, :],
                  pag_send_r.at[i], pag_recv_r.at[i], p_right)
        sl_b = pl.ds(ag_bots[s] * PCH + STRIP + hoff, 64)
        rl = push(cbuf.at[sl_b, :], cbuf.at[sl_b, :],
                  pag_send_l.at[i], pag_recv_l.at[i], p_left)
        return rr, rl

    def ag_wait(pair, s, hoff):
        rr, rl = pair
        rr.wait_recv()
        sl = pl.ds(rc_tops[s] * PCH + hoff, 64)
        out_ref[sl, :] = cbuf[sl, :].astype(f32)
        rl.wait_recv()
        sl = pl.ds(rc_bots[s] * PCH + STRIP + hoff, 64)
        out_ref[sl, :] = cbuf[sl, :].astype(f32)

    a0 = ag_send(0, 0, hoff0)
    rds = []
    for si, base in enumerate(bases):
        sl = pl.ds(base + hoff0, 64)
        rds.append(push(cbuf.at[sl, :], cbuf.at[sl, :],
                        z_send.at[6 + si], z_recv.at[6 + si], z1p))
    sl = pl.ds(base_r + hoff0, 64)
    out_ref[sl, :] = cbuf[sl, :].astype(f32)
    sl = pl.ds(base_l + hoff0, 64)
    out_ref[sl, :] = cbuf[sl, :].astype(f32)
    for rd in rds:
        rd.wait_recv()
    c0 = ag_send(1, 0, hoff1)
    sl = pl.ds(base_r + hoff1, 64)
    out_ref[sl, :] = cbuf[sl, :].astype(f32)
    sl = pl.ds(base_l + hoff1, 64)
    out_ref[sl, :] = cbuf[sl, :].astype(f32)
    ag_wait(a0, 0, hoff0)
    a1 = ag_send(0, 1, hoff0)
    ag_wait(c0, 0, hoff1)
    c1 = ag_send(1, 1, hoff1)
    ag_wait(a1, 1, hoff0)
    a2 = ag_send(0, 2, hoff0)
    ag_wait(c1, 1, hoff1)
    c2 = ag_send(1, 2, hoff1)
    ag_wait(a2, 2, hoff0)
    ag_wait(c2, 2, hoff1)

    for rd in sends:
        rd.wait_send()


def kernel(x, Wq, K_ext, V_ext, Wo):
    x2 = x.reshape(SQ, SQ)
    k2 = K_ext.reshape(SKV, D_LOC)
    v2 = V_ext.reshape(SKV, D_LOC)

    out = pl.pallas_call(
        _body,
        out_shape=jax.ShapeDtypeStruct((SQ, SQ), jnp.float32),
        in_specs=[
            pl.BlockSpec(memory_space=pltpu.VMEM),
            pl.BlockSpec(memory_space=pltpu.MemorySpace.HBM),
            pl.BlockSpec(memory_space=pltpu.VMEM),
            pl.BlockSpec(memory_space=pltpu.VMEM),
            pl.BlockSpec(memory_space=pltpu.MemorySpace.HBM),
        ],
        out_specs=pl.BlockSpec(memory_space=pltpu.VMEM),
        scratch_shapes=(
            [
                pltpu.VMEM((SQ, D_LOC), jnp.float32),
                pltpu.VMEM((D_LOC, SQ), jnp.float32),
                pltpu.VMEM((SQ, SQ), jnp.bfloat16),
                pltpu.VMEM((SQ, SKV), jnp.bfloat16),
                pltpu.VMEM((3, STRIP, SQ), jnp.bfloat16),
                pltpu.VMEM((3, STRIP, SQ), jnp.bfloat16),
                pltpu.VMEM((2, 64, SQ), jnp.bfloat16),
                pltpu.VMEM((2, 32, SQ), jnp.bfloat16),
                pltpu.SemaphoreType.DMA,
                pltpu.SemaphoreType.DMA,
            ]
            + [pltpu.SemaphoreType.DMA((3,))] * 4
            + [pltpu.SemaphoreType.DMA((8,))] * 2
            + [pltpu.SemaphoreType.DMA((6,))] * 4
        ),
        compiler_params=pltpu.CompilerParams(collective_id=0),
    )(x2, Wq, k2, v2, Wo)
    return out.reshape(1, SQ, SQ)
